# Initial kernel scaffold; baseline (speedup 1.0000x reference)
import jax
import jax.numpy as jnp
from jax.experimental import pallas as pl


def kernel(X, theta_W, theta_b, a_src, a_dst, v_idx, e_idx, e2v_src, e2v_dst):
    raise NotImplementedError("write your pallas kernel here")


# speedup vs baseline: 10.5391x; 10.5391x over previous
import jax
import jax.numpy as jnp
from jax import lax
from jax.experimental import pallas as pl
from jax.experimental.pallas import tpu as pltpu
from jax.experimental.pallas import tpu_sc as plsc

N = 10000
M = 5000
P = 320000
D = 128

L = 16
NC = 2
NS = 16
KW = 128
NWIN = 79
PT = NWIN * KW
PP = NC * NS * PT
NPV = 10240
NPE = 5120

f32 = jnp.float32
i32 = jnp.int32

_CP = pltpu.CompilerParams(needs_layout_passes=False)


def _mesh():
    return plsc.VectorSubcoreMesh(core_axis_name="c", subcore_axis_name="s",
                                  num_cores=NC, num_subcores=NS)


def _nrsqrt(x):
    i = plsc.bitcast(x, i32)
    y = plsc.bitcast(jnp.int32(0x5F3759DF) - (i >> 1), f32)
    for _ in range(3):
        y = y * (1.5 - 0.5 * x * y * y)
    return y


def _wid():
    return lax.axis_index("c") * NS + lax.axis_index("s")


def _blocks(w, total_blocks, nworkers):
    lo = total_blocks // nworkers
    hi_workers = total_blocks - lo * nworkers
    nb = jnp.where(w < hi_workers, lo + 1, lo)
    b0 = jnp.where(w < hi_workers, (lo + 1) * w,
                   hi_workers * (lo + 1) + lo * (w - hi_workers))
    return b0, nb, (lo + 1 if hi_workers else lo)


def _zero_rows(rowsa, nrows):
    def z(j, _):
        for g in range(D // L):
            rowsa[j, pl.ds(g * L, L)] = jnp.zeros((L,), f32)
        return 0
    lax.fori_loop(0, nrows, z, 0)


def _tc_body(x_ref, w_ref, b_ref, a2_ref, xt_ref, sv_ref):
    xt = lax.dot_general(x_ref[...], w_ref[...],
                         (((1,), (1,)), ((), ())),
                         preferred_element_type=f32)
    xt = xt + b_ref[...]
    xt_ref[...] = xt
    sv_ref[...] = lax.dot_general(xt, a2_ref[...],
                                  (((1,), (0,)), ((), ())),
                                  preferred_element_type=f32)


def _s0_body(xs_ref, xd_ref, vi_ref, ei_ref, es_ref, ed_ref,
             ex_ref, degv_ref, dege_ref, denom_ref,
             degv, dege, denom,
             vidx_t, eidx_t, esrc_t, edst_t, ex_t, bufa, bufb, ones_b, zb):
    c = lax.axis_index("c")
    s = lax.axis_index("s")
    w = c * NS + s
    vstripe = NPV // NS
    estripe = NPE // NS
    vb = s * vstripe
    eb = s * estripe

    pltpu.sync_copy(vi_ref.at[w], vidx_t)
    pltpu.sync_copy(ei_ref.at[w], eidx_t)
    pltpu.sync_copy(es_ref.at[w], esrc_t)
    pltpu.sync_copy(ed_ref.at[w], edst_t)

    def zfill(k, _):
        zb[pl.ds(k * L, L)] = jnp.zeros((L,), f32)
        return 0
    lax.fori_loop(0, vstripe // L, zfill, 0)
    for g in range(KW // L):
        ones_b[pl.ds(g * L, L)] = jnp.ones((L,), f32)
    pltpu.sync_copy(zb.at[pl.ds(0, vstripe)], degv.at[pl.ds(vb, vstripe)])
    pltpu.sync_copy(zb.at[pl.ds(0, vstripe)], denom.at[pl.ds(vb, vstripe)])
    zb0, znb, zmax = _blocks(s, NPE // KW, NS)
    for bi in range(zmax):
        @pl.when(bi < znb)
        def _():
            pltpu.sync_copy(zb.at[pl.ds(0, KW)],
                            dege.at[pl.ds((zb0 + bi) * KW, KW)])
    plsc.subcore_barrier()

    def bwin(wi, _):
        pltpu.sync_copy(xs_ref.at[esrc_t.at[wi]], bufa)
        pltpu.sync_copy(xd_ref.at[edst_t.at[wi]], bufb)
        for g in range(KW // L):
            sl = pl.ds(g * L, L)
            sv = bufa[sl] + bufb[sl]
            sv = jnp.maximum(sv, 0.2 * sv)
            sv = jnp.minimum(jnp.maximum(sv, 0.001), 5.0)
            ex_t[wi, sl] = jnp.exp(sv)
        pltpu.sync_copy(ex_t.at[wi], denom.at[vidx_t.at[wi]], add=True)
        pltpu.sync_copy(ones_b, degv.at[vidx_t.at[wi]], add=True)
        pltpu.sync_copy(ones_b, dege.at[eidx_t.at[wi]], add=True)
        return 0
    lax.fori_loop(0, NWIN, bwin, 0)

    pltpu.sync_copy(ex_t, ex_ref.at[w])
    plsc.subcore_barrier()
    pltpu.sync_copy(degv.at[pl.ds(vb, vstripe)],
                    degv_ref.at[c].at[pl.ds(vb, vstripe)])
    pltpu.sync_copy(denom.at[pl.ds(vb, vstripe)],
                    denom_ref.at[c].at[pl.ds(vb, vstripe)])
    b0, nb, maxnb = _blocks(s, NPE // KW, NS)
    for bi in range(maxnb):
        @pl.when(bi < nb)
        def _():
            blk = (b0 + bi) * KW
            pltpu.sync_copy(dege.at[pl.ds(blk, KW)],
                            dege_ref.at[c].at[pl.ds(blk, KW)])


def _make_s0():
    return pl.kernel(
    _s0_body,
    out_type=(jax.ShapeDtypeStruct((NC * NS, NWIN, KW), f32),
              jax.ShapeDtypeStruct((NC, NPV), f32),
              jax.ShapeDtypeStruct((NC, NPE), f32),
              jax.ShapeDtypeStruct((NC, NPV), f32)),
    mesh=_mesh(),
    scratch_types=[
        pltpu.MemorySpace.VMEM_SHARED((NPV,), f32),
        pltpu.MemorySpace.VMEM_SHARED((NPE,), f32),
        pltpu.MemorySpace.VMEM_SHARED((NPV,), f32),
        pltpu.VMEM((NWIN, KW), i32), pltpu.VMEM((NWIN, KW), i32),
        pltpu.VMEM((NWIN, KW), i32), pltpu.VMEM((NWIN, KW), i32),
        pltpu.VMEM((NWIN, KW), f32),
        pltpu.VMEM((KW,), f32), pltpu.VMEM((KW,), f32),
        pltpu.VMEM((KW,), f32), pltpu.VMEM((NPV // NS,), f32),
    ],
    compiler_params=_CP)


def _s1_body(xt_ref, degv_ref, dege_ref, denom_ref,
             vtab_ref, isdv_ref, invdv_ref, invdn_ref, invde_ref,
             sa, sb, sc_, rowsa):
    w = _wid()
    NW = NC * NS

    def vblock(blk):
        r0 = blk * KW
        pltpu.sync_copy(degv_ref.at[0].at[pl.ds(r0, KW)], sa)
        pltpu.sync_copy(degv_ref.at[1].at[pl.ds(r0, KW)], sb)

        def cv(k, _):
            sl = pl.ds(k * L, L)
            d = sa[sl] + sb[sl]
            pos = d > 0.0
            dm = jnp.maximum(d, 1.0)
            sc_[sl] = jnp.where(pos, _nrsqrt(dm), 0.0)
            sb[sl] = jnp.where(pos, 1.0 / dm, 0.0)
            return 0
        lax.fori_loop(0, KW // L, cv, 0)
        pltpu.sync_copy(sc_, isdv_ref.at[pl.ds(r0, KW)])
        pltpu.sync_copy(sb, invdv_ref.at[pl.ds(r0, KW)])

        pltpu.sync_copy(denom_ref.at[0].at[pl.ds(r0, KW)], sa)
        pltpu.sync_copy(denom_ref.at[1].at[pl.ds(r0, KW)], sb)

        def cn(k, _):
            sl = pl.ds(k * L, L)
            dn = sa[sl] + sb[sl]
            sa[sl] = jnp.where(dn > 0.0, 1.0 / jnp.maximum(dn, 1e-30), 0.0)
            return 0
        lax.fori_loop(0, KW // L, cn, 0)
        pltpu.sync_copy(sa, invdn_ref.at[pl.ds(r0, KW)])

        pltpu.sync_copy(xt_ref.at[pl.ds(r0, KW)], rowsa)

        def rw(j, _):
            f = plsc.load_gather(sc_, [jnp.zeros((L,), i32) + j])
            for g in range(D // L):
                sl = pl.ds(g * L, L)
                rowsa[j, sl] = rowsa[j, sl] * f
            return 0
        lax.fori_loop(0, KW, rw, 0)
        pltpu.sync_copy(rowsa, vtab_ref.at[pl.ds(r0, KW)])

    def eblock(blk):
        r0 = blk * KW
        pltpu.sync_copy(dege_ref.at[0].at[pl.ds(r0, KW)], sa)
        pltpu.sync_copy(dege_ref.at[1].at[pl.ds(r0, KW)], sb)

        def ce(k, _):
            sl = pl.ds(k * L, L)
            d = sa[sl] + sb[sl]
            sa[sl] = jnp.where(d > 0.0, 1.0 / jnp.maximum(d, 1.0), 0.0)
            return 0
        lax.fori_loop(0, KW // L, ce, 0)
        pltpu.sync_copy(sa, invde_ref.at[pl.ds(r0, KW)])

    vb0, vnb, vmax = _blocks(w, NPV // KW, NW)
    for bi in range(vmax):
        @pl.when(bi < vnb)
        def _():
            vblock(vb0 + bi)
    eb0, enb, emax = _blocks(w, NPE // KW, NW)
    for bi in range(emax):
        @pl.when(bi < enb)
        def _():
            eblock(eb0 + bi)


def _make_s1():
    return pl.kernel(
    _s1_body,
    out_type=(jax.ShapeDtypeStruct((NPV, D), f32),
              jax.ShapeDtypeStruct((NPV,), f32),
              jax.ShapeDtypeStruct((NPV,), f32),
              jax.ShapeDtypeStruct((NPV,), f32),
              jax.ShapeDtypeStruct((NPE,), f32)),
    mesh=_mesh(),
    scratch_types=[
        pltpu.VMEM((KW,), f32),
        pltpu.VMEM((KW,), f32),
        pltpu.VMEM((KW,), f32),
        pltpu.VMEM((KW, D), f32),
    ],
    compiler_params=_CP)


def _make_scatter(trows, weighted):
    tstripe = trows // NS

    if weighted:
        def body(src_ref, si_ref, di_ref, ex_ref, idn_ref, part_ref,
                 acc, sidx_t, didx_t, rowsa, ex_t, ibuf):
            _scatter_common(src_ref, si_ref, di_ref, part_ref, acc,
                            sidx_t, didx_t, rowsa, tstripe,
                            ex_ref=ex_ref, idn_ref=idn_ref,
                            ex_t=ex_t, ibuf=ibuf)
        scratch = [
            pltpu.MemorySpace.VMEM_SHARED((trows, D), f32),
            pltpu.VMEM((NWIN, KW), i32), pltpu.VMEM((NWIN, KW), i32),
            pltpu.VMEM((KW, D), f32),
            pltpu.VMEM((NWIN, KW), f32), pltpu.VMEM((KW,), f32),
        ]
        intypes = 5
    else:
        def body(src_ref, si_ref, di_ref, part_ref,
                 acc, sidx_t, didx_t, rowsa):
            _scatter_common(src_ref, si_ref, di_ref, part_ref, acc,
                            sidx_t, didx_t, rowsa, tstripe)
        scratch = [
            pltpu.MemorySpace.VMEM_SHARED((trows, D), f32),
            pltpu.VMEM((NWIN, KW), i32), pltpu.VMEM((NWIN, KW), i32),
            pltpu.VMEM((KW, D), f32),
        ]
        intypes = 3
    del intypes
    return pl.kernel(
        body,
        out_type=jax.ShapeDtypeStruct((NC, trows, D), f32),
        mesh=_mesh(), scratch_types=scratch, compiler_params=_CP)


def _scatter_common(src_ref, si_ref, di_ref, part_ref, acc,
                    sidx_t, didx_t, rowsa, tstripe,
                    ex_ref=None, idn_ref=None, ex_t=None, ibuf=None):
    c = lax.axis_index("c")
    s = lax.axis_index("s")
    w = c * NS + s
    tb = s * tstripe

    pltpu.sync_copy(si_ref.at[w], sidx_t)
    pltpu.sync_copy(di_ref.at[w], didx_t)
    if ex_ref is not None:
        pltpu.sync_copy(ex_ref.at[w], ex_t)

    _zero_rows(rowsa, 64)
    nzc = tstripe // 64
    for ci in range(nzc):
        pltpu.sync_copy(rowsa.at[pl.ds(0, 64)],
                        acc.at[pl.ds(tb + ci * 64, 64)])
    plsc.subcore_barrier()

    def pw(wi, _):
        pltpu.sync_copy(src_ref.at[sidx_t.at[wi]], rowsa)
        if ex_ref is not None:
            pltpu.sync_copy(idn_ref.at[didx_t.at[wi]], ibuf)
            for g in range(KW // L):
                sl = pl.ds(g * L, L)
                ibuf[sl] = ex_t[wi, sl] * ibuf[sl]

            def rw(j, _):
                f = plsc.load_gather(ibuf, [jnp.zeros((L,), i32) + j])
                for g in range(D // L):
                    sl = pl.ds(g * L, L)
                    rowsa[j, sl] = rowsa[j, sl] * f
                return 0
            lax.fori_loop(0, KW, rw, 0)
        pltpu.sync_copy(rowsa, acc.at[didx_t.at[wi]], add=True)
        return 0
    lax.fori_loop(0, NWIN, pw, 0)
    plsc.subcore_barrier()

    for ci in range(tstripe // 64):
        r0 = tb + ci * 64
        pltpu.sync_copy(acc.at[pl.ds(r0, 64)],
                        part_ref.at[c].at[pl.ds(r0, 64)])


def _make_merge(trows, scaled, elu):
    def compute(part_ref, fac_ref, tab_ref, bufa, bufb, fbuf):
        w = _wid()

        def block(blk):
            r0 = blk * KW
            if scaled:
                pltpu.sync_copy(fac_ref.at[pl.ds(r0, KW)], fbuf)
            pltpu.sync_copy(part_ref.at[0].at[pl.ds(r0, KW)], bufa)
            pltpu.sync_copy(part_ref.at[1].at[pl.ds(r0, KW)], bufb)

            def rw(j, _):
                if scaled:
                    f = plsc.load_gather(fbuf, [jnp.zeros((L,), i32) + j])
                for g in range(D // L):
                    sl = pl.ds(g * L, L)
                    x = bufa[j, sl] + bufb[j, sl]
                    if scaled:
                        x = x * f
                    if elu:
                        x = jnp.where(x > 0.0, x, jnp.exp(x) - 1.0)
                    bufa[j, sl] = x
                return 0
            lax.fori_loop(0, KW, rw, 0)
            pltpu.sync_copy(bufa, tab_ref.at[pl.ds(r0, KW)])

        b0, nb, maxnb = _blocks(w, trows // KW, NC * NS)
        for bi in range(maxnb):
            @pl.when(bi < nb)
            def _():
                block(b0 + bi)

    if scaled:
        def body(part_ref, fac_ref, tab_ref, bufa, bufb, fbuf):
            compute(part_ref, fac_ref, tab_ref, bufa, bufb, fbuf)
    else:
        def body(part_ref, tab_ref, bufa, bufb, fbuf):
            compute(part_ref, None, tab_ref, bufa, bufb, fbuf)

    return pl.kernel(
        body,
        out_type=jax.ShapeDtypeStruct((trows, D), f32),
        mesh=_mesh(),
        scratch_types=[
            pltpu.VMEM((KW, D), f32), pltpu.VMEM((KW, D), f32),
            pltpu.VMEM((KW,), f32),
        ],
        compiler_params=_CP)


_CACHE = {}


def _kernels():
    if not _CACHE:
        _CACHE.update(
            s0=_make_s0(), s1=_make_s1(),
            scv=_make_scatter(NPV, weighted=False),
            scvw=_make_scatter(NPV, weighted=True),
            sce=_make_scatter(NPE, weighted=False),
            mgv=_make_merge(NPV, scaled=True, elu=False),
            mgvp=_make_merge(NPV, scaled=False, elu=False),
            mgve=_make_merge(NPV, scaled=True, elu=True),
            mge=_make_merge(NPE, scaled=True, elu=False),
        )
    return _CACHE


def kernel(X, theta_W, theta_b, a_src, a_dst, v_idx, e_idx, e2v_src, e2v_dst):
    Xp = jnp.concatenate([X, jnp.zeros((NPV - N, D), f32)], axis=0)
    b2 = theta_b.reshape(1, D)
    A2 = jnp.stack([a_src, a_dst] + [jnp.zeros((D,), f32)] * 6, axis=1)

    tc = pl.pallas_call(
        _tc_body,
        out_shape=[jax.ShapeDtypeStruct((NPV, D), f32),
                   jax.ShapeDtypeStruct((NPV, 8), f32)],
    )
    xtH, sv = tc(Xp, theta_W, b2, A2)
    xsp = sv[:, 0]
    xdp = sv[:, 1]

    npad = PP - P
    ar = jnp.arange(npad, dtype=i32)
    shape3 = (NC * NS, NWIN, KW)
    vI = jnp.concatenate([v_idx.astype(i32), N + ar % (NPV - N)]).reshape(shape3)
    eI = jnp.concatenate([e_idx.astype(i32), M + ar % (NPE - M)]).reshape(shape3)
    sI = jnp.concatenate([e2v_src.astype(i32), jnp.zeros((npad,), i32)]).reshape(shape3)
    dI = jnp.concatenate([e2v_dst.astype(i32), jnp.zeros((npad,), i32)]).reshape(shape3)

    K = _kernels()
    exH, degvP, degeP, denomP = K["s0"](xsp, xdp, vI, eI, sI, dI)
    vtabH, isdvH, invdvH, invdnH, invdeH = K["s1"](xtH, degvP, degeP, denomP)

    ep = K["sce"](vtabH, vI, eI)
    etab = K["mge"](ep, invdeH)
    vp = K["scv"](etab, eI, vI)
    vtab2 = K["mgv"](vp, isdvH)
    ep = K["sce"](vtab2, vI, eI)
    etab2 = K["mge"](ep, invdeH)
    vp = K["scvw"](etab2, eI, vI, exH, invdnH)
    vtab3 = K["mgvp"](vp)
    ep = K["sce"](vtab3, vI, eI)
    etab3 = K["mge"](ep, invdeH)
    vp = K["scv"](etab3, eI, vI)
    out = K["mgve"](vp, invdvH)

    return out[:N]
